# baseline (speedup 1.0000x reference)
import functools

import jax
import jax.numpy as jnp
from jax.experimental import pallas as pl
from jax.experimental.pallas import tpu as pltpu


def _mm_body(x_ref, w_ref, o_ref):
    o_ref[...] = jnp.dot(x_ref[...], w_ref[...],
                         preferred_element_type=jnp.float32)


def _mm(x, w, bn=1000):
    n, k = x.shape
    _, m = w.shape
    return pl.pallas_call(
        _mm_body,
        grid=(n // bn,),
        in_specs=[pl.BlockSpec((bn, k), lambda i: (i, 0)),
                  pl.BlockSpec((k, m), lambda i: (0, 0))],
        out_specs=pl.BlockSpec((bn, m), lambda i: (i, 0)),
        out_shape=jax.ShapeDtypeStruct((n, m), jnp.float32),
    )(x, w)


def kernel(x, edge_index, edge_weight, W0, b0, W1, b1, W2, b2, W3, b3):
    n = x.shape[0]
    src = edge_index[0]
    dst = edge_index[1]

    def spmm(s):
        msgs = s[src] * edge_weight[:, None]
        return jax.ops.segment_sum(msgs, dst, num_segments=n)

    s0 = _mm(x, W0)
    h0 = jax.nn.relu(spmm(s0) + b0)
    s1 = _mm(h0, W1)
    h1 = jax.nn.relu(spmm(s1) + b1)
    s2 = _mm(h1, W2)
    h2 = jax.nn.relu(spmm(s2) + b2) + h0
    s3 = _mm(h2, W3)
    out = spmm(s3) + b3
    return jax.nn.log_softmax(out, axis=1)


# speedup vs baseline: 2.1306x; 2.1306x over previous
import functools

import jax
import jax.numpy as jnp
from jax import lax
from jax.experimental import pallas as pl
from jax.experimental.pallas import tpu as pltpu
from jax.experimental.pallas import tpu_sc as plsc

N_SC = 2
N_TILE = 16
LANES = 16
CHUNK = 128


def _mm0_body(x_ref, w_ref, oL_ref, oR_ref):
    r = jnp.dot(x_ref[...], w_ref[...], preferred_element_type=jnp.float32)
    oL_ref[...] = r[:, :128]
    oR_ref[...] = r[:, 128:]


def _mm0(x, w, bn=1000):
    n, k = x.shape
    half = w.shape[1] // 2
    return pl.pallas_call(
        _mm0_body,
        grid=(n // bn,),
        in_specs=[pl.BlockSpec((bn, k), lambda i: (i, 0)),
                  pl.BlockSpec((k, 2 * half), lambda i: (0, 0))],
        out_specs=[pl.BlockSpec((bn, half), lambda i: (i, 0)),
                   pl.BlockSpec((bn, half), lambda i: (i, 0))],
        out_shape=[jax.ShapeDtypeStruct((n, half), jnp.float32),
                   jax.ShapeDtypeStruct((n, half), jnp.float32)],
    )(x, w)


def _mm_mid_body(aL_ref, aR_ref, b_ref, w_ref, oL_ref, oR_ref):
    h = jnp.concatenate([aL_ref[...], aR_ref[...]], axis=1) + b_ref[...]
    h = jnp.maximum(h, 0.0)
    r = jnp.dot(h, w_ref[...], preferred_element_type=jnp.float32)
    oL_ref[...] = r[:, :128]
    oR_ref[...] = r[:, 128:]


def _mm_mid(aL, aR, b, w, bn=1000):
    n, half = aL.shape
    k = 2 * half
    return pl.pallas_call(
        _mm_mid_body,
        grid=(n // bn,),
        in_specs=[pl.BlockSpec((bn, half), lambda i: (i, 0)),
                  pl.BlockSpec((bn, half), lambda i: (i, 0)),
                  pl.BlockSpec((1, k), lambda i: (0, 0)),
                  pl.BlockSpec((k, k), lambda i: (0, 0))],
        out_specs=[pl.BlockSpec((bn, half), lambda i: (i, 0)),
                   pl.BlockSpec((bn, half), lambda i: (i, 0))],
        out_shape=[jax.ShapeDtypeStruct((n, half), jnp.float32),
                   jax.ShapeDtypeStruct((n, half), jnp.float32)],
    )(aL, aR, b.reshape(1, k), w)


def _mm_last_body(a2L_ref, a2R_ref, b2_ref, a0L_ref, a0R_ref, b0_ref,
                  w_ref, o_ref):
    h2 = jnp.concatenate([a2L_ref[...], a2R_ref[...]], axis=1) + b2_ref[...]
    h0 = jnp.concatenate([a0L_ref[...], a0R_ref[...]], axis=1) + b0_ref[...]
    h = jnp.maximum(h2, 0.0) + jnp.maximum(h0, 0.0)
    o_ref[...] = jnp.dot(h, w_ref[...], preferred_element_type=jnp.float32)


def _mm_last(a2L, a2R, b2, a0L, a0R, b0, wp, bn=1000):
    n, half = a2L.shape
    k = 2 * half
    m = wp.shape[1]
    return pl.pallas_call(
        _mm_last_body,
        grid=(n // bn,),
        in_specs=[pl.BlockSpec((bn, half), lambda i: (i, 0)),
                  pl.BlockSpec((bn, half), lambda i: (i, 0)),
                  pl.BlockSpec((1, k), lambda i: (0, 0)),
                  pl.BlockSpec((bn, half), lambda i: (i, 0)),
                  pl.BlockSpec((bn, half), lambda i: (i, 0)),
                  pl.BlockSpec((1, k), lambda i: (0, 0)),
                  pl.BlockSpec((k, m), lambda i: (0, 0))],
        out_specs=pl.BlockSpec((bn, m), lambda i: (i, 0)),
        out_shape=jax.ShapeDtypeStruct((n, m), jnp.float32),
    )(a2L, a2R, b2.reshape(1, k), a0L, a0R, b0.reshape(1, k), wp)


def _lsm_body(nclass, p0_ref, p1_ref, b_ref, o_ref):
    z = p0_ref[...] + p1_ref[...] + b_ref[...]
    mpad = z.shape[1]
    col = lax.broadcasted_iota(jnp.int32, z.shape, 1)
    z = jnp.where(col < nclass, z, -1e30)
    m = jnp.max(z, axis=1, keepdims=True)
    lse = jnp.log(jnp.sum(jnp.exp(z - m), axis=1, keepdims=True))
    o_ref[...] = (z - m - lse)[:, :nclass]


def _log_softmax(p0, p1, bp, nclass, bn=1000):
    n, mpad = p0.shape
    return pl.pallas_call(
        functools.partial(_lsm_body, nclass),
        grid=(n // bn,),
        in_specs=[pl.BlockSpec((bn, mpad), lambda i: (i, 0)),
                  pl.BlockSpec((bn, mpad), lambda i: (i, 0)),
                  pl.BlockSpec((1, mpad), lambda i: (0, 0))],
        out_specs=pl.BlockSpec((bn, nclass), lambda i: (i, 0)),
        out_shape=jax.ShapeDtypeStruct((n, nclass), jnp.float32),
    )(p0, p1, bp.reshape(1, mpad))


def _zero_rows(rows_v, nrows, half):
    zero = jnp.zeros((LANES,), jnp.float32)
    nvec = half // LANES

    def body(i, _):
        r = i // nvec
        j = i % nvec
        rows_v[r, pl.ds(j * LANES, LANES)] = zero
        return 0

    lax.fori_loop(0, nrows * nvec, body, 0)


def _scale_rows(rows_v, w_v, half):
    nvec = half // LANES

    def body(g, _):
        wvec = w_v[pl.ds(g * LANES, LANES)]
        for l in range(LANES):
            e = g * LANES + l
            wl = wvec[l]
            for j in range(nvec):
                sl = pl.ds(j * LANES, LANES)
                rows_v[e, sl] = rows_v[e, sl] * wl
        return 0

    lax.fori_loop(0, CHUNK // LANES, body, 0)


def _copy_stripe(src_ref, dst_ref, base, nrows):
    for off in range(0, nrows, CHUNK):
        sz = min(CHUNK, nrows - off)
        pltpu.sync_copy(src_ref.at[pl.ds(base + off, sz)],
                        dst_ref.at[pl.ds(base + off, sz)])


def _zero_stripe(rows_v, acc, base, nrows):
    for off in range(0, nrows, CHUNK):
        sz = min(CHUNK, nrows - off)
        pltpu.sync_copy(rows_v.at[pl.ds(0, sz)],
                        acc.at[pl.ds(base + off, sz)])


def _tile_stripe(n, sid):
    stride = (n // N_TILE // 8) * 8
    last = n - (N_TILE - 1) * stride
    base = pl.multiple_of(sid * stride, 8)
    return base, stride, last


def _spmm_featsplit(supL, supR, src, dst, w, n):
    half = supL.shape[1]
    e_pad = src.shape[0]
    ept = e_pad // N_TILE
    nch = ept // CHUNK
    rpt = n // N_TILE
    nvec = half // LANES
    mesh = plsc.VectorSubcoreMesh(core_axis_name="c", subcore_axis_name="s")

    @functools.partial(
        pl.kernel,
        out_type=[jax.ShapeDtypeStruct((n, half), jnp.float32),
                  jax.ShapeDtypeStruct((n, half), jnp.float32)],
        mesh=mesh,
        scratch_types=[
            pltpu.VMEM_SHARED((n, half), jnp.float32),
            pltpu.VMEM((CHUNK,), jnp.int32),
            pltpu.VMEM((CHUNK,), jnp.int32),
            pltpu.VMEM((CHUNK,), jnp.float32),
            pltpu.VMEM((CHUNK, half), jnp.float32),
            pltpu.SemaphoreType.DMA,
        ],
    )
    def k(supL_hbm, supR_hbm, src_hbm, dst_hbm, w_hbm, outL_hbm, outR_hbm,
          acc, src_v, dst_v, w_v, rows_v, sem):
        cid = lax.axis_index("c")
        sid = lax.axis_index("s")

        _zero_rows(rows_v, CHUNK, half)
        base, stride, last_sz = _tile_stripe(n, sid)

        @pl.when(sid < N_TILE - 1)
        def _():
            _zero_stripe(rows_v, acc, base, stride)

        @pl.when(sid == N_TILE - 1)
        def _():
            _zero_stripe(rows_v, acc, base, last_sz)

        plsc.subcore_barrier()

        ebase = sid * ept

        def chunk_body(c, _):
            off = ebase + c * CHUNK
            pltpu.sync_copy(src_hbm.at[pl.ds(off, CHUNK)], src_v)
            pltpu.sync_copy(dst_hbm.at[pl.ds(off, CHUNK)], dst_v)
            pltpu.sync_copy(w_hbm.at[pl.ds(off, CHUNK)], w_v)

            @pl.when(cid == 0)
            def _():
                pltpu.async_copy(supL_hbm.at[src_v], rows_v, sem).wait()

            @pl.when(cid == 1)
            def _():
                pltpu.async_copy(supR_hbm.at[src_v], rows_v, sem).wait()

            _scale_rows(rows_v, w_v, half)
            pltpu.sync_copy(rows_v, acc.at[dst_v], add=True)
            return 0

        lax.fori_loop(0, nch, chunk_body, 0)
        plsc.subcore_barrier()

        for c, out_hbm in ((0, outL_hbm), (1, outR_hbm)):
            @pl.when((cid == c) & (sid < N_TILE - 1))
            def _():
                _copy_stripe(acc, out_hbm, base, stride)

            @pl.when((cid == c) & (sid == N_TILE - 1))
            def _():
                _copy_stripe(acc, out_hbm, base, last_sz)

    return k(supL, supR, src, dst, w)


def _spmm_edgesplit(sup, src, dst, w, n):
    width = sup.shape[1]
    e_pad = src.shape[0]
    ept = e_pad // (N_SC * N_TILE)
    nch = ept // CHUNK
    rpt = n // N_TILE
    mesh = plsc.VectorSubcoreMesh(core_axis_name="c", subcore_axis_name="s")

    @functools.partial(
        pl.kernel,
        out_type=[jax.ShapeDtypeStruct((n, width), jnp.float32),
                  jax.ShapeDtypeStruct((n, width), jnp.float32)],
        mesh=mesh,
        scratch_types=[
            pltpu.VMEM_SHARED((n, width), jnp.float32),
            pltpu.VMEM((CHUNK,), jnp.int32),
            pltpu.VMEM((CHUNK,), jnp.int32),
            pltpu.VMEM((CHUNK,), jnp.float32),
            pltpu.VMEM((CHUNK, width), jnp.float32),
            pltpu.SemaphoreType.DMA,
        ],
    )
    def k(sup_hbm, src_hbm, dst_hbm, w_hbm, out0_hbm, out1_hbm,
          acc, src_v, dst_v, w_v, rows_v, sem):
        cid = lax.axis_index("c")
        sid = lax.axis_index("s")

        _zero_rows(rows_v, CHUNK, width)
        base, stride, last_sz = _tile_stripe(n, sid)

        @pl.when(sid < N_TILE - 1)
        def _():
            _zero_stripe(rows_v, acc, base, stride)

        @pl.when(sid == N_TILE - 1)
        def _():
            _zero_stripe(rows_v, acc, base, last_sz)

        plsc.subcore_barrier()

        ebase = (cid * N_TILE + sid) * ept

        def chunk_body(c, _):
            off = ebase + c * CHUNK
            pltpu.sync_copy(src_hbm.at[pl.ds(off, CHUNK)], src_v)
            pltpu.sync_copy(dst_hbm.at[pl.ds(off, CHUNK)], dst_v)
            pltpu.sync_copy(w_hbm.at[pl.ds(off, CHUNK)], w_v)
            pltpu.async_copy(sup_hbm.at[src_v], rows_v, sem).wait()
            _scale_rows(rows_v, w_v, width)
            pltpu.sync_copy(rows_v, acc.at[dst_v], add=True)
            return 0

        lax.fori_loop(0, nch, chunk_body, 0)
        plsc.subcore_barrier()

        for c, out_hbm in ((0, out0_hbm), (1, out1_hbm)):
            @pl.when((cid == c) & (sid < N_TILE - 1))
            def _():
                _copy_stripe(acc, out_hbm, base, stride)

            @pl.when((cid == c) & (sid == N_TILE - 1))
            def _():
                _copy_stripe(acc, out_hbm, base, last_sz)

    return k(sup, src, dst, w)


def kernel(x, edge_index, edge_weight, W0, b0, W1, b1, W2, b2, W3, b3):
    n = x.shape[0]
    e = edge_index.shape[1]
    nclass = W3.shape[1]

    gran = N_SC * N_TILE * CHUNK
    e_pad = ((e + gran - 1) // gran) * gran
    src = jnp.pad(edge_index[0].astype(jnp.int32), (0, e_pad - e))
    dst = jnp.pad(edge_index[1].astype(jnp.int32), (0, e_pad - e))
    w = jnp.pad(edge_weight, (0, e_pad - e))

    mpad = ((nclass + 127) // 128) * 128
    W3p = jnp.pad(W3, ((0, 0), (0, mpad - nclass)))
    b3p = jnp.pad(b3, (0, mpad - nclass))

    s0L, s0R = _mm0(x, W0)
    a0L, a0R = _spmm_featsplit(s0L, s0R, src, dst, w, n)
    s1L, s1R = _mm_mid(a0L, a0R, b0, W1)
    a1L, a1R = _spmm_featsplit(s1L, s1R, src, dst, w, n)
    s2L, s2R = _mm_mid(a1L, a1R, b1, W2)
    a2L, a2R = _spmm_featsplit(s2L, s2R, src, dst, w, n)
    s3 = _mm_last(a2L, a2R, b2, a0L, a0R, b0, W3p)
    p0, p1 = _spmm_edgesplit(s3, src, dst, w, n)
    return _log_softmax(p0, p1, b3p, nclass)


# speedup vs baseline: 2.9591x; 1.3889x over previous
import functools

import jax
import jax.numpy as jnp
from jax import lax
from jax.experimental import pallas as pl
from jax.experimental.pallas import tpu as pltpu
from jax.experimental.pallas import tpu_sc as plsc

N_SC = 2
N_TILE = 16
LANES = 16
CHUNK = 128


def _mm0_body(x_ref, w_ref, oL_ref, oR_ref):
    r = jnp.dot(x_ref[...], w_ref[...], preferred_element_type=jnp.float32)
    oL_ref[...] = r[:, :128]
    oR_ref[...] = r[:, 128:]


def _mm0(x, w, bn=1000):
    n, k = x.shape
    half = w.shape[1] // 2
    return pl.pallas_call(
        _mm0_body,
        grid=(n // bn,),
        in_specs=[pl.BlockSpec((bn, k), lambda i: (i, 0)),
                  pl.BlockSpec((k, 2 * half), lambda i: (0, 0))],
        out_specs=[pl.BlockSpec((bn, half), lambda i: (i, 0)),
                   pl.BlockSpec((bn, half), lambda i: (i, 0))],
        out_shape=[jax.ShapeDtypeStruct((n, half), jnp.float32),
                   jax.ShapeDtypeStruct((n, half), jnp.float32)],
    )(x, w)


def _mm_mid_body(aL_ref, aR_ref, b_ref, w_ref, oL_ref, oR_ref):
    h = jnp.concatenate([aL_ref[...], aR_ref[...]], axis=1) + b_ref[...]
    h = jnp.maximum(h, 0.0)
    r = jnp.dot(h, w_ref[...], preferred_element_type=jnp.float32)
    oL_ref[...] = r[:, :128]
    oR_ref[...] = r[:, 128:]


def _mm_mid(aL, aR, b, w, bn=1000):
    n, half = aL.shape
    k = 2 * half
    return pl.pallas_call(
        _mm_mid_body,
        grid=(n // bn,),
        in_specs=[pl.BlockSpec((bn, half), lambda i: (i, 0)),
                  pl.BlockSpec((bn, half), lambda i: (i, 0)),
                  pl.BlockSpec((1, k), lambda i: (0, 0)),
                  pl.BlockSpec((k, k), lambda i: (0, 0))],
        out_specs=[pl.BlockSpec((bn, half), lambda i: (i, 0)),
                   pl.BlockSpec((bn, half), lambda i: (i, 0))],
        out_shape=[jax.ShapeDtypeStruct((n, half), jnp.float32),
                   jax.ShapeDtypeStruct((n, half), jnp.float32)],
    )(aL, aR, b.reshape(1, k), w)


def _mm_last_body(a2L_ref, a2R_ref, b2_ref, a0L_ref, a0R_ref, b0_ref,
                  w_ref, o_ref):
    h2 = jnp.concatenate([a2L_ref[...], a2R_ref[...]], axis=1) + b2_ref[...]
    h0 = jnp.concatenate([a0L_ref[...], a0R_ref[...]], axis=1) + b0_ref[...]
    h = jnp.maximum(h2, 0.0) + jnp.maximum(h0, 0.0)
    o_ref[...] = jnp.dot(h, w_ref[...], preferred_element_type=jnp.float32)


def _mm_last(a2L, a2R, b2, a0L, a0R, b0, wp, bn=1000):
    n, half = a2L.shape
    k = 2 * half
    m = wp.shape[1]
    return pl.pallas_call(
        _mm_last_body,
        grid=(n // bn,),
        in_specs=[pl.BlockSpec((bn, half), lambda i: (i, 0)),
                  pl.BlockSpec((bn, half), lambda i: (i, 0)),
                  pl.BlockSpec((1, k), lambda i: (0, 0)),
                  pl.BlockSpec((bn, half), lambda i: (i, 0)),
                  pl.BlockSpec((bn, half), lambda i: (i, 0)),
                  pl.BlockSpec((1, k), lambda i: (0, 0)),
                  pl.BlockSpec((k, m), lambda i: (0, 0))],
        out_specs=pl.BlockSpec((bn, m), lambda i: (i, 0)),
        out_shape=jax.ShapeDtypeStruct((n, m), jnp.float32),
    )(a2L, a2R, b2.reshape(1, k), a0L, a0R, b0.reshape(1, k), wp)


def _lsm_body(nclass, p0_ref, p1_ref, b_ref, o_ref):
    z = p0_ref[...] + p1_ref[...] + b_ref[...]
    mpad = z.shape[1]
    col = lax.broadcasted_iota(jnp.int32, z.shape, 1)
    z = jnp.where(col < nclass, z, -1e30)
    m = jnp.max(z, axis=1, keepdims=True)
    lse = jnp.log(jnp.sum(jnp.exp(z - m), axis=1, keepdims=True))
    o_ref[...] = (z - m - lse)[:, :nclass]


def _log_softmax(p0, p1, bp, nclass, bn=1000):
    n, mpad = p0.shape
    return pl.pallas_call(
        functools.partial(_lsm_body, nclass),
        grid=(n // bn,),
        in_specs=[pl.BlockSpec((bn, mpad), lambda i: (i, 0)),
                  pl.BlockSpec((bn, mpad), lambda i: (i, 0)),
                  pl.BlockSpec((1, mpad), lambda i: (0, 0))],
        out_specs=pl.BlockSpec((bn, nclass), lambda i: (i, 0)),
        out_shape=jax.ShapeDtypeStruct((n, nclass), jnp.float32),
    )(p0, p1, bp.reshape(1, mpad))


def _zero_rows(rows_v, nrows, half):
    zero = jnp.zeros((LANES,), jnp.float32)
    nvec = half // LANES

    def body(i, _):
        r = i // nvec
        j = i % nvec
        rows_v[r, pl.ds(j * LANES, LANES)] = zero
        return 0

    lax.fori_loop(0, nrows * nvec, body, 0)


def _scale_rows(rows_v, w_v, half):
    nvec = half // LANES

    def body(g, _):
        wvec = w_v[pl.ds(g * LANES, LANES)]
        for l in range(LANES):
            e = g * LANES + l
            wl = wvec[l]
            for j in range(nvec):
                sl = pl.ds(j * LANES, LANES)
                rows_v[e, sl] = rows_v[e, sl] * wl
        return 0

    lax.fori_loop(0, CHUNK // LANES, body, 0)


def _copy_stripe(src_ref, dst_ref, base, nrows):
    for off in range(0, nrows, CHUNK):
        sz = min(CHUNK, nrows - off)
        pltpu.sync_copy(src_ref.at[pl.ds(base + off, sz)],
                        dst_ref.at[pl.ds(base + off, sz)])


def _zero_stripe(rows_v, acc, base, nrows):
    for off in range(0, nrows, CHUNK):
        sz = min(CHUNK, nrows - off)
        pltpu.sync_copy(rows_v.at[pl.ds(0, sz)],
                        acc.at[pl.ds(base + off, sz)])


def _tile_stripe(n, sid):
    stride = (n // N_TILE // 8) * 8
    last = n - (N_TILE - 1) * stride
    base = pl.multiple_of(sid * stride, 8)
    return base, stride, last


def _spmm_sc(supL, supR, idx2, w2d, n, split_edges):
    half = supL.shape[1]
    tot_ch = idx2.shape[0]
    nch = tot_ch // (N_SC * N_TILE if split_edges else N_TILE)
    mesh = plsc.VectorSubcoreMesh(core_axis_name="c", subcore_axis_name="s")

    @functools.partial(
        pl.kernel,
        out_type=[jax.ShapeDtypeStruct((n, half), jnp.float32),
                  jax.ShapeDtypeStruct((n, half), jnp.float32)],
        mesh=mesh,
        scratch_types=[
            pltpu.VMEM_SHARED((n, half), jnp.float32),
            pltpu.VMEM((2, CHUNK), jnp.int32),
            pltpu.VMEM((2, CHUNK), jnp.int32),
            pltpu.VMEM((CHUNK,), jnp.float32),
            pltpu.VMEM((CHUNK,), jnp.float32),
            pltpu.VMEM((CHUNK, half), jnp.float32),
            pltpu.VMEM((CHUNK, half), jnp.float32),
            pltpu.SemaphoreType.DMA,
            pltpu.SemaphoreType.DMA,
            pltpu.SemaphoreType.DMA,
            pltpu.SemaphoreType.DMA,
            pltpu.SemaphoreType.DMA,
            pltpu.SemaphoreType.DMA,
        ],
    )
    def k(supL_hbm, supR_hbm, idx_hbm, w_hbm, outL_hbm, outR_hbm,
          acc, idx0, idx1, wv0, wv1, rows0, rows1,
          gsem0, gsem1, isem0, isem1, wsem0, wsem1):
        cid = lax.axis_index("c")
        sid = lax.axis_index("s")
        rows = (rows0, rows1)
        idxs = (idx0, idx1)
        wvs = (wv0, wv1)
        gsems = (gsem0, gsem1)
        isems = (isem0, isem1)
        wsems = (wsem0, wsem1)

        _zero_rows(rows0, CHUNK, half)
        base, stride, last_sz = _tile_stripe(n, sid)

        @pl.when(sid < N_TILE - 1)
        def _():
            _zero_stripe(rows0, acc, base, stride)

        @pl.when(sid == N_TILE - 1)
        def _():
            _zero_stripe(rows0, acc, base, last_sz)

        plsc.subcore_barrier()

        if split_edges:
            cb = (cid * N_TILE + sid) * nch
        else:
            cb = sid * nch

        def start_idx(c, b):
            pltpu.make_async_copy(idx_hbm.at[cb + c], idxs[b],
                                  isems[b]).start()
            pltpu.make_async_copy(w_hbm.at[cb + c], wvs[b],
                                  wsems[b]).start()

        def wait_idx(c, b):
            pltpu.make_async_copy(idx_hbm.at[cb + c], idxs[b],
                                  isems[b]).wait()
            pltpu.make_async_copy(w_hbm.at[cb + c], wvs[b],
                                  wsems[b]).wait()

        def start_gather(c, b):
            @pl.when(cid == 0)
            def _():
                pltpu.make_async_copy(
                    supL_hbm.at[idxs[b].at[0]], rows[b], gsems[b]).start()

            @pl.when(cid == 1)
            def _():
                pltpu.make_async_copy(
                    supR_hbm.at[idxs[b].at[0]], rows[b], gsems[b]).start()

        def wait_gather(c, b):
            @pl.when(cid == 0)
            def _():
                pltpu.make_async_copy(
                    supL_hbm.at[idxs[b].at[0]], rows[b], gsems[b]).wait()

            @pl.when(cid == 1)
            def _():
                pltpu.make_async_copy(
                    supR_hbm.at[idxs[b].at[0]], rows[b], gsems[b]).wait()

        start_idx(0, 0)
        wait_idx(0, 0)
        start_gather(0, 0)
        start_idx(1, 1)

        def pair_body(p, _):
            for b in (0, 1):
                c = p * 2 + b
                nb = 1 - b
                wait_gather(c, b)

                @pl.when(c + 1 < nch)
                def _():
                    wait_idx(c + 1, nb)
                    start_gather(c + 1, nb)

                _scale_rows(rows[b], wvs[b], half)
                pltpu.sync_copy(rows[b], acc.at[idxs[b].at[1]], add=True)

                @pl.when(c + 2 < nch)
                def _():
                    start_idx(c + 2, b)
            return 0

        lax.fori_loop(0, nch // 2, pair_body, 0)
        plsc.subcore_barrier()

        for c, out_hbm in ((0, outL_hbm), (1, outR_hbm)):
            @pl.when((cid == c) & (sid < N_TILE - 1))
            def _():
                _copy_stripe(acc, out_hbm, base, stride)

            @pl.when((cid == c) & (sid == N_TILE - 1))
            def _():
                _copy_stripe(acc, out_hbm, base, last_sz)

    return k(supL, supR, idx2, w2d)


def kernel(x, edge_index, edge_weight, W0, b0, W1, b1, W2, b2, W3, b3):
    n = x.shape[0]
    e = edge_index.shape[1]
    nclass = W3.shape[1]

    gran = 2 * N_SC * N_TILE * CHUNK
    e_pad = ((e + gran - 1) // gran) * gran
    src = jnp.pad(edge_index[0].astype(jnp.int32),
                  (0, e_pad - e)).reshape(e_pad // CHUNK, CHUNK)
    dst = jnp.pad(edge_index[1].astype(jnp.int32),
                  (0, e_pad - e)).reshape(e_pad // CHUNK, CHUNK)
    w = jnp.pad(edge_weight, (0, e_pad - e)).reshape(e_pad // CHUNK, CHUNK)
    idx2 = jnp.stack([src, dst], axis=1)

    mpad = ((nclass + 127) // 128) * 128
    W3p = jnp.pad(W3, ((0, 0), (0, mpad - nclass)))
    b3p = jnp.pad(b3, (0, mpad - nclass))

    s0L, s0R = _mm0(x, W0)
    a0L, a0R = _spmm_sc(s0L, s0R, idx2, w, n, split_edges=False)
    s1L, s1R = _mm_mid(a0L, a0R, b0, W1)
    a1L, a1R = _spmm_sc(s1L, s1R, idx2, w, n, split_edges=False)
    s2L, s2R = _mm_mid(a1L, a1R, b1, W2)
    a2L, a2R = _spmm_sc(s2L, s2R, idx2, w, n, split_edges=False)
    s3 = _mm_last(a2L, a2R, b2, a0L, a0R, b0, W3p)
    p0, p1 = _spmm_sc(s3, s3, idx2, w, n, split_edges=True)
    return _log_softmax(p0, p1, b3p, nclass)
